# Initial kernel scaffold; baseline (speedup 1.0000x reference)
import jax
import jax.numpy as jnp
from jax.experimental import pallas as pl


def kernel(x, edge_index, W1, b1, W2, b2):
    raise NotImplementedError("write your pallas kernel here")


# speedup vs baseline: 9.8860x; 9.8860x over previous
import functools

import jax
import jax.numpy as jnp
from jax import lax
from jax.experimental import pallas as pl
from jax.experimental.pallas import tpu as pltpu
from jax.experimental.pallas import tpu_sc as plsc

N = 10000
E = 320000
F_IN = 128
HID = 128
NCLS = 64

NC = 2
NS = 16
LANES = 16

NPAD = 10240
TRASHN = NPAD - N
E2 = 327680
EPT = E2 // NS
ECH = 128
NCHUNK = EPT // ECH
NBUF = 4
NGRP = NCHUNK // NBUF
NPT = NPAD // NS
NODE_CH = 40
NNCH = NPT // NODE_CH
CRING = 2 * NBUF

EPW = E2 // (NC * NS)
PCH = 2048
PROWS = PCH // 128

_mesh = plsc.VectorSubcoreMesh(core_axis_name="c", subcore_axis_name="s")


def _f32(x):
    return jnp.asarray(x, jnp.float32)


@functools.partial(
    pl.kernel,
    out_type=(
        jax.ShapeDtypeStruct((NC * NPAD,), jnp.float32),
        jax.ShapeDtypeStruct((E2 // 128, 128), jnp.int32),
    ),
    mesh=_mesh,
    scratch_types=dict(
        rbuf=pltpu.VMEM((PROWS, 128), jnp.int32),
        cbuf=pltpu.VMEM((PROWS, 128), jnp.int32),
        cpbuf=pltpu.VMEM((PROWS, 128), jnp.int32),
        wbuf=pltpu.VMEM((PROWS, 128), jnp.float32),
        zbuf=pltpu.VMEM((NPT,), jnp.float32),
        degacc=pltpu.VMEM_SHARED((NPAD,), jnp.float32),
        sem=pltpu.SemaphoreType.DMA,
    ),
)
def _edge_prep(row_hbm, col_hbm, deg_hbm, colp_hbm, rbuf, cbuf, cpbuf, wbuf,
               zbuf, degacc, sem):
    c = lax.axis_index("c")
    s = lax.axis_index("s")
    wid = c * NS + s

    def _z(t, _):
        zbuf[pl.ds(t * LANES, LANES)] = jnp.zeros((LANES,), jnp.float32)
        return _
    lax.fori_loop(0, NPT // LANES, _z, None)
    pltpu.sync_copy(zbuf, degacc.at[pl.ds(s * NPT, NPT)])
    plsc.subcore_barrier()

    def chunk(ch, _):
        base = wid * (EPW // 128) + ch * PROWS
        pltpu.sync_copy(row_hbm.at[pl.ds(base, PROWS)], rbuf)
        pltpu.sync_copy(col_hbm.at[pl.ds(base, PROWS)], cbuf)

        def vec(t, _):
            j = t // (128 // LANES)
            k = t % (128 // LANES)
            r = rbuf[j, pl.ds(k * LANES, LANES)]
            cc = cbuf[j, pl.ds(k * LANES, LANES)]
            m = r != cc
            cpbuf[j, pl.ds(k * LANES, LANES)] = jnp.where(
                m, cc, N + jnp.remainder(cc, TRASHN))
            wbuf[j, pl.ds(k * LANES, LANES)] = jnp.where(m, 1.0, 0.0)
            return _
        lax.fori_loop(0, PROWS * (128 // LANES), vec, None)

        pltpu.sync_copy(cpbuf, colp_hbm.at[pl.ds(base, PROWS)])
        for j in range(PROWS):
            pltpu.sync_copy(wbuf.at[j], degacc.at[rbuf.at[j]], add=True)
        return _
    lax.fori_loop(0, EPW // PCH, chunk, None)
    plsc.subcore_barrier()

    pltpu.sync_copy(degacc.at[pl.ds(s * NPT, NPT)], zbuf)
    pltpu.sync_copy(zbuf, deg_hbm.at[pl.ds(c * NPAD + s * NPT, NPT)])


def _make_prop(W):
    QN = W // LANES

    @functools.partial(
        pl.kernel,
        out_type=(
            jax.ShapeDtypeStruct((NC * NPAD, W), jnp.float32),
            jax.ShapeDtypeStruct((NC * NPAD, W), jnp.float32),
        ),
        mesh=_mesh,
        scratch_types=dict(
            rixb=pltpu.VMEM((NBUF, ECH), jnp.int32),
            cixb=pltpu.VMEM((CRING, ECH), jnp.int32),
            gbuf=pltpu.VMEM((NBUF, ECH, W), jnp.float32),
            nacc=pltpu.VMEM((NODE_CH, W), jnp.float32),
            nsy=pltpu.VMEM((NODE_CH, W), jnp.float32),
            nu=pltpu.VMEM((NODE_CH, W), jnp.float32),
            zbuf=pltpu.VMEM((NODE_CH, W), jnp.float32),
            s2b=pltpu.VMEM((NPT,), jnp.float32),
            acc_sh=pltpu.VMEM_SHARED((NPAD, W), jnp.float32),
            gsem=pltpu.SemaphoreType.DMA((NBUF,)),
            ssem=pltpu.SemaphoreType.DMA((CRING,)),
            risem=pltpu.SemaphoreType.DMA((NBUF,)),
            cisem=pltpu.SemaphoreType.DMA((CRING,)),
        ),
        compiler_params=pltpu.CompilerParams(needs_layout_passes=False,
                                             use_tc_tiling_on_sc=False),
    )
    def prop(rowoff_hbm, colp_hbm, u3_hbm, sy2_hbm, sy1_hbm, s2_hbm, m_hbm,
             uw_hbm, rixb, cixb, gbuf, nacc, nsy, nu, zbuf, s2b, acc_sh,
             gsem, ssem, risem, cisem):
        c = lax.axis_index("c")
        s = lax.axis_index("s")
        nbase = s * NPT
        ebase = s * NCHUNK
        row_hbm = rowoff_hbm.at[c]

        pltpu.sync_copy(s2_hbm.at[pl.ds(nbase, NPT)], s2b)

        def _z(t, _):
            def _zrow(q, _2):
                zbuf[t, pl.ds(q * LANES, LANES)] = jnp.zeros((LANES,),
                                                             jnp.float32)
                return _2
            lax.fori_loop(0, QN, _zrow, None)
            return _
        lax.fori_loop(0, NODE_CH, _z, None)

        for q in range(NNCH):
            r0 = nbase + q * NODE_CH
            pltpu.sync_copy(zbuf, acc_sh.at[pl.ds(r0, NODE_CH)])
        plsc.subcore_barrier()

        def row_load(i, b):
            pltpu.async_copy(row_hbm.at[pl.ds(ebase + i, 1)],
                             rixb.at[pl.ds(b, 1)], risem.at[b])

        def col_load(i, b):
            pltpu.async_copy(colp_hbm.at[pl.ds(ebase + i, 1)],
                             cixb.at[pl.ds(b, 1)], cisem.at[b])

        def sweep(u_hbm):
            for b in range(CRING):
                col_load(b, b)
            for b in range(NBUF):
                row_load(b, b)

            def group(g, _):
                for b in range(CRING):
                    i = g * CRING + b
                    gb = b % NBUF
                    ob = (b + NBUF) % CRING
                    pltpu.make_async_copy(
                        row_hbm.at[pl.ds(ebase + i, 1)],
                        rixb.at[pl.ds(gb, 1)], risem.at[gb]).wait()

                    @pl.when(i >= NBUF)
                    def _():
                        pltpu.make_async_copy(
                            gbuf.at[gb], acc_sh.at[cixb.at[ob]],
                            ssem.at[ob]).wait()

                        @pl.when(i + NBUF < NCHUNK)
                        def _():
                            col_load(i + NBUF, ob)

                    pltpu.async_copy(u_hbm.at[rixb.at[gb]], gbuf.at[gb],
                                     gsem.at[gb])
                    pltpu.make_async_copy(u_hbm.at[rixb.at[gb]], gbuf.at[gb],
                                          gsem.at[gb]).wait()

                    @pl.when(i + NBUF < NCHUNK)
                    def _():
                        row_load(i + NBUF, gb)

                    pltpu.make_async_copy(
                        colp_hbm.at[pl.ds(ebase + i, 1)],
                        cixb.at[pl.ds(b, 1)], cisem.at[b]).wait()
                    pltpu.async_copy(gbuf.at[gb], acc_sh.at[cixb.at[b]],
                                     ssem.at[b], add=True)
                return _
            lax.fori_loop(0, NCHUNK // CRING, group, None)
            for k in range(NBUF):
                i = NCHUNK - NBUF + k
                pltpu.make_async_copy(gbuf.at[k % NBUF],
                                      acc_sh.at[cixb.at[i % CRING]],
                                      ssem.at[i % CRING]).wait()
            plsc.subcore_barrier()

        def rescale(sy_hbm):
            for q in range(NNCH):
                r0 = nbase + q * NODE_CH
                pltpu.sync_copy(acc_sh.at[pl.ds(r0, NODE_CH)], nacc)
                pltpu.sync_copy(sy_hbm.at[pl.ds(c * NPAD + r0, NODE_CH)], nsy)

                def node(j, _):
                    jj = q * NODE_CH + j
                    s2v = plsc.load_gather(
                        s2b, [jnp.full((LANES,), jj, jnp.int32)])
                    for qq in range(QN):
                        sl = pl.ds(qq * LANES, LANES)
                        nu[j, sl] = nsy[j, sl] + s2v * nacc[j, sl]
                    return _
                lax.fori_loop(0, NODE_CH, node, None)

                pltpu.sync_copy(nu, uw_hbm.at[pl.ds(c * NPAD + r0, NODE_CH)])
                pltpu.sync_copy(zbuf, acc_sh.at[pl.ds(r0, NODE_CH)])
            plsc.subcore_barrier()

        sweep(u3_hbm)
        rescale(sy2_hbm)
        sweep(uw_hbm)
        rescale(sy1_hbm)
        sweep(uw_hbm)

        for q in range(NNCH):
            r0 = nbase + q * NODE_CH
            pltpu.sync_copy(acc_sh.at[pl.ds(r0, NODE_CH)], nacc)
            pltpu.sync_copy(nacc, m_hbm.at[pl.ds(c * NPAD + r0, NODE_CH)])

    return prop


_prop64 = _make_prop(64)
_prop32 = _make_prop(32)


_BN = 128
_GRID = NPAD // _BN


def _tc_prep_body(x_ref, w_ref, deg_ref, y0_ref, u3_ref, sy2_ref, sy1_ref,
                  s_ref, s2_ref):
    deg = deg_ref[0, :] + deg_ref[1, :]
    sv = jnp.where(deg > 0, lax.rsqrt(deg), 0.0)[:, None]
    y = jnp.dot(x_ref[...], w_ref[...], preferred_element_type=jnp.float32)
    y0_ref[...] = y[:, :HID]
    sy1_ref[...] = sv * y[:, HID:2 * HID]
    sy2_ref[...] = sv * y[:, 2 * HID:3 * HID]
    u3_ref[...] = sv * y[:, 3 * HID:]
    s_ref[...] = sv
    s2_ref[...] = sv * sv


def _tc_prep(xp, wcat, degpart):
    f32 = jnp.float32
    outs = (
        jax.ShapeDtypeStruct((NPAD, HID), f32),
        jax.ShapeDtypeStruct((NPAD, HID), f32),
        jax.ShapeDtypeStruct((NPAD, HID), f32),
        jax.ShapeDtypeStruct((NPAD, HID), f32),
        jax.ShapeDtypeStruct((NPAD, 1), f32),
        jax.ShapeDtypeStruct((NPAD, 1), f32),
    )
    blk = pl.BlockSpec((_BN, HID), lambda i: (i, 0))
    blk1 = pl.BlockSpec((_BN, 1), lambda i: (i, 0))
    return pl.pallas_call(
        _tc_prep_body,
        grid=(_GRID,),
        in_specs=[
            pl.BlockSpec((_BN, F_IN), lambda i: (i, 0)),
            pl.BlockSpec((F_IN, 4 * HID), lambda i: (0, 0)),
            pl.BlockSpec((NC, _BN), lambda i: (0, i)),
        ],
        out_specs=[blk, blk, blk, blk, blk1, blk1],
        out_shape=outs,
    )(xp, wcat, degpart)


def _tc_mid_body(m1_ref, y0_ref, s_ref, b1_ref, w_ref, z0_ref, u3_ref,
                 sz2_ref, sz1_ref):
    sv = s_ref[...]
    h = jnp.maximum(y0_ref[...] + sv * m1_ref[...] + b1_ref[...], 0.0)
    z = jnp.dot(h, w_ref[...], preferred_element_type=jnp.float32)
    z0_ref[...] = z[:, :NCLS]
    sz1_ref[...] = sv * z[:, NCLS:2 * NCLS]
    sz2_ref[...] = sv * z[:, 2 * NCLS:3 * NCLS]
    u3_ref[...] = sv * z[:, 3 * NCLS:]


def _tc_mid(m1, y0, svec, b1, wcat):
    f32 = jnp.float32
    outs = tuple(jax.ShapeDtypeStruct((NPAD, NCLS), f32) for _ in range(4))
    blk = pl.BlockSpec((_BN, NCLS), lambda i: (i, 0))
    return pl.pallas_call(
        _tc_mid_body,
        grid=(_GRID,),
        in_specs=[
            pl.BlockSpec((_BN, HID), lambda i: (i, 0)),
            pl.BlockSpec((_BN, HID), lambda i: (i, 0)),
            pl.BlockSpec((_BN, 1), lambda i: (i, 0)),
            pl.BlockSpec((1, HID), lambda i: (0, 0)),
            pl.BlockSpec((HID, 4 * NCLS), lambda i: (0, 0)),
        ],
        out_specs=[blk, blk, blk, blk],
        out_shape=outs,
    )(m1, y0, svec, b1, wcat)


def _tc_post_body(z0_ref, m2_ref, s_ref, b2_ref, out_ref):
    o = z0_ref[...] + s_ref[...] * m2_ref[...] + b2_ref[...]
    mx = jnp.max(o, axis=1, keepdims=True)
    ex = jnp.exp(o - mx)
    lse = mx + jnp.log(jnp.sum(ex, axis=1, keepdims=True))
    out_ref[...] = o - lse


def _tc_post(z0, m2, svec, b2):
    return pl.pallas_call(
        _tc_post_body,
        grid=(_GRID,),
        in_specs=[
            pl.BlockSpec((_BN, NCLS), lambda i: (i, 0)),
            pl.BlockSpec((_BN, NCLS), lambda i: (i, 0)),
            pl.BlockSpec((_BN, 1), lambda i: (i, 0)),
            pl.BlockSpec((1, NCLS), lambda i: (0, 0)),
        ],
        out_specs=pl.BlockSpec((_BN, NCLS), lambda i: (i, 0)),
        out_shape=jax.ShapeDtypeStruct((NPAD, NCLS), jnp.float32),
    )(z0, m2, svec, b2)


def _split_cols(a, w):
    return a.reshape(NPAD, 2, w).transpose(1, 0, 2).reshape(2 * NPAD, w)


def _merge_cols(a, w):
    return a.reshape(2, NPAD, w).transpose(1, 0, 2).reshape(NPAD, 2 * w)


def _fold(W):
    return jnp.concatenate(
        [W[0] - W[2], 3.0 * W[3] - W[1], 2.0 * W[2], -4.0 * W[3]], axis=1)


def kernel(x, edge_index, W1, b1, W2, b2):
    x = _f32(x)
    wcat1 = _fold(_f32(W1))
    wcat2 = _fold(_f32(W2))

    row = edge_index[0].astype(jnp.int32)
    col = edge_index[1].astype(jnp.int32)
    padv = (jnp.arange(E, E2, dtype=jnp.int32)) % N
    rowp = jnp.concatenate([row, padv]).reshape(E2 // 128, 128)
    colp_in = jnp.concatenate([col, padv]).reshape(E2 // 128, 128)
    rowoff = jnp.stack([rowp, rowp + NPAD])

    degpart, colp = _edge_prep(rowp, colp_in)
    degpart = degpart.reshape(NC, NPAD)

    xp = jnp.pad(x, ((0, NPAD - N), (0, 0)))
    y0, u3, sy2, sy1, svec, s2vec = _tc_prep(xp, wcat1, degpart)
    s2flat = s2vec.reshape(NPAD)

    m1s, _ = _prop64(rowoff, colp, _split_cols(u3, 64), _split_cols(sy2, 64),
                     _split_cols(sy1, 64), s2flat)
    m1 = _merge_cols(m1s, 64)

    z0, u3z, sz2, sz1 = _tc_mid(m1, y0, svec, b1.reshape(1, HID), wcat2)

    m2s, _ = _prop32(rowoff, colp, _split_cols(u3z, 32), _split_cols(sz2, 32),
                     _split_cols(sz1, 32), s2flat)
    m2 = _merge_cols(m2s, 32)

    out = _tc_post(z0, m2, svec, b2.reshape(1, NCLS))
    return out[:N]


# speedup vs baseline: 12.1266x; 1.2266x over previous
import functools

import jax
import jax.numpy as jnp
from jax import lax
from jax.experimental import pallas as pl
from jax.experimental.pallas import tpu as pltpu
from jax.experimental.pallas import tpu_sc as plsc

N = 10000
E = 320000
F_IN = 128
HID = 128
NCLS = 64

NC = 2
NS = 16
LANES = 16

NPAD = 10240
TRASHN = NPAD - N
E2 = 327680
EPT = E2 // NS
ECH = 128
NCHUNK = EPT // ECH
NBUF = 4
NGRP = NCHUNK // NBUF
NPT = NPAD // NS
NODE_CH = 40
NNCH = NPT // NODE_CH
CRING = 2 * NBUF

EPW = E2 // (NC * NS)
PCH = 2048
PROWS = PCH // 128

_mesh = plsc.VectorSubcoreMesh(core_axis_name="c", subcore_axis_name="s")


def _f32(x):
    return jnp.asarray(x, jnp.float32)


@functools.partial(
    pl.kernel,
    out_type=(
        jax.ShapeDtypeStruct((NC * NPAD,), jnp.float32),
        jax.ShapeDtypeStruct((E2 // 128, 128), jnp.int32),
    ),
    mesh=_mesh,
    scratch_types=dict(
        rbuf=pltpu.VMEM((PROWS, 128), jnp.int32),
        cbuf=pltpu.VMEM((PROWS, 128), jnp.int32),
        cpbuf=pltpu.VMEM((PROWS, 128), jnp.int32),
        wbuf=pltpu.VMEM((PROWS, 128), jnp.float32),
        zbuf=pltpu.VMEM((NPT,), jnp.float32),
        degacc=pltpu.VMEM_SHARED((NPAD,), jnp.float32),
        sem=pltpu.SemaphoreType.DMA,
    ),
)
def _edge_prep(row_hbm, col_hbm, deg_hbm, colp_hbm, rbuf, cbuf, cpbuf, wbuf,
               zbuf, degacc, sem):
    c = lax.axis_index("c")
    s = lax.axis_index("s")
    wid = c * NS + s

    def _z(t, _):
        zbuf[pl.ds(t * LANES, LANES)] = jnp.zeros((LANES,), jnp.float32)
        return _
    lax.fori_loop(0, NPT // LANES, _z, None)
    pltpu.sync_copy(zbuf, degacc.at[pl.ds(s * NPT, NPT)])
    plsc.subcore_barrier()

    def chunk(ch, _):
        base = wid * (EPW // 128) + ch * PROWS
        pltpu.sync_copy(row_hbm.at[pl.ds(base, PROWS)], rbuf)
        pltpu.sync_copy(col_hbm.at[pl.ds(base, PROWS)], cbuf)

        def vec(t, _):
            j = t // (128 // LANES)
            k = t % (128 // LANES)
            r = rbuf[j, pl.ds(k * LANES, LANES)]
            cc = cbuf[j, pl.ds(k * LANES, LANES)]
            m = r != cc
            cpbuf[j, pl.ds(k * LANES, LANES)] = jnp.where(
                m, cc, N + jnp.remainder(cc, TRASHN))
            wbuf[j, pl.ds(k * LANES, LANES)] = jnp.where(m, 1.0, 0.0)
            return _
        lax.fori_loop(0, PROWS * (128 // LANES), vec, None)

        pltpu.sync_copy(cpbuf, colp_hbm.at[pl.ds(base, PROWS)])
        for j in range(PROWS):
            pltpu.sync_copy(wbuf.at[j], degacc.at[rbuf.at[j]], add=True)
        return _
    lax.fori_loop(0, EPW // PCH, chunk, None)
    plsc.subcore_barrier()

    pltpu.sync_copy(degacc.at[pl.ds(s * NPT, NPT)], zbuf)
    pltpu.sync_copy(zbuf, deg_hbm.at[pl.ds(c * NPAD + s * NPT, NPT)])


def _make_prop(W):
    QN = W // LANES

    @functools.partial(
        pl.kernel,
        out_type=(
            jax.ShapeDtypeStruct((NC * NPAD, W), jnp.float32),
            jax.ShapeDtypeStruct((NC * NPAD, W), jnp.float32),
        ),
        mesh=_mesh,
        scratch_types=dict(
            rixb=pltpu.VMEM((2, NBUF, ECH), jnp.int32),
            cixb=pltpu.VMEM((2, NBUF, ECH), jnp.int32),
            gbuf=pltpu.VMEM((NBUF, ECH, W), jnp.float32),
            nsy=pltpu.VMEM((NODE_CH, W), jnp.float32),
            nu=pltpu.VMEM((NODE_CH, W), jnp.float32),
            zbuf=pltpu.VMEM((NODE_CH, W), jnp.float32),
            s2b=pltpu.VMEM((NPT,), jnp.float32),
            acc_sh=pltpu.VMEM_SHARED((NPAD, W), jnp.float32),
            gsem=pltpu.SemaphoreType.DMA((NBUF,)),
            ssem=pltpu.SemaphoreType.DMA((NBUF,)),
            risem=pltpu.SemaphoreType.DMA((2,)),
            cisem=pltpu.SemaphoreType.DMA((2,)),
            asem=pltpu.SemaphoreType.DMA,
            bsem=pltpu.SemaphoreType.DMA,
        ),
        compiler_params=pltpu.CompilerParams(needs_layout_passes=False,
                                             use_tc_tiling_on_sc=False),
    )
    def prop(rowoff_hbm, colp_hbm, u3_hbm, sy2_hbm, sy1_hbm, s2_hbm, m_hbm,
             uw_hbm, rixb, cixb, gbuf, nsy, nu, zbuf, s2b, acc_sh,
             gsem, ssem, risem, cisem, asem, bsem):
        c = lax.axis_index("c")
        s = lax.axis_index("s")
        nbase = s * NPT
        ebase = s * NCHUNK
        row_hbm = rowoff_hbm.at[c]

        pltpu.sync_copy(s2_hbm.at[pl.ds(nbase, NPT)], s2b)

        def _z(t, _):
            def _zrow(q, _2):
                zbuf[t, pl.ds(q * LANES, LANES)] = jnp.zeros((LANES,),
                                                             jnp.float32)
                return _2
            lax.fori_loop(0, QN, _zrow, None)
            return _
        lax.fori_loop(0, NODE_CH, _z, None)

        for q in range(NNCH):
            r0 = nbase + q * NODE_CH
            pltpu.sync_copy(zbuf, acc_sh.at[pl.ds(r0, NODE_CH)])
        plsc.subcore_barrier()

        def idx_load(g, p):
            pltpu.async_copy(row_hbm.at[pl.ds(ebase + g * NBUF, NBUF)],
                             rixb.at[p], risem.at[p])
            pltpu.async_copy(colp_hbm.at[pl.ds(ebase + g * NBUF, NBUF)],
                             cixb.at[p], cisem.at[p])

        def idx_wait(g, p):
            pltpu.make_async_copy(row_hbm.at[pl.ds(ebase + g * NBUF, NBUF)],
                                  rixb.at[p], risem.at[p]).wait()
            pltpu.make_async_copy(colp_hbm.at[pl.ds(ebase + g * NBUF, NBUF)],
                                  cixb.at[p], cisem.at[p]).wait()

        def sweep(u_hbm):
            idx_load(0, 0)

            def do_group(g, p, first, last):
                rix = rixb.at[p]
                cix = cixb.at[p]
                idx_wait(g, p)
                if not first:
                    for b in range(NBUF):
                        pltpu.make_async_copy(
                            gbuf.at[b], acc_sh.at[cixb.at[1 - p].at[b]],
                            ssem.at[b]).wait()
                if not last:
                    idx_load(g + 1, 1 - p)
                for b in range(NBUF):
                    pltpu.async_copy(u_hbm.at[rix.at[b]], gbuf.at[b],
                                     gsem.at[b])
                for b in range(NBUF):
                    pltpu.make_async_copy(u_hbm.at[rix.at[b]], gbuf.at[b],
                                          gsem.at[b]).wait()
                    pltpu.async_copy(gbuf.at[b], acc_sh.at[cix.at[b]],
                                     ssem.at[b], add=True)

            do_group(0, 0, True, False)

            def pair(G, _):
                g0 = 2 * G + 1
                do_group(g0, 1, False, False)
                do_group(g0 + 1, 0, False, False)
                return _
            lax.fori_loop(0, (NGRP - 2) // 2, pair, None)
            do_group(NGRP - 1, 1, False, True)
            for b in range(NBUF):
                pltpu.make_async_copy(gbuf.at[b], acc_sh.at[cixb.at[1].at[b]],
                                      ssem.at[b]).wait()
            plsc.subcore_barrier()

        def rescale(sy_hbm):
            for q in range(NNCH):
                r0 = nbase + q * NODE_CH
                pltpu.async_copy(acc_sh.at[pl.ds(r0, NODE_CH)], nu, asem)
                pltpu.async_copy(sy_hbm.at[pl.ds(c * NPAD + r0, NODE_CH)],
                                 nsy, bsem)
                pltpu.make_async_copy(acc_sh.at[pl.ds(r0, NODE_CH)], nu,
                                      asem).wait()
                pltpu.make_async_copy(sy_hbm.at[pl.ds(c * NPAD + r0, NODE_CH)],
                                      nsy, bsem).wait()

                def node(j, _):
                    jj = q * NODE_CH + j
                    s2v = plsc.load_gather(
                        s2b, [jnp.full((LANES,), jj, jnp.int32)])
                    for qq in range(QN):
                        sl = pl.ds(qq * LANES, LANES)
                        nu[j, sl] = nsy[j, sl] + s2v * nu[j, sl]
                    return _
                lax.fori_loop(0, NODE_CH, node, None)

                pltpu.sync_copy(nu, uw_hbm.at[pl.ds(c * NPAD + r0, NODE_CH)])
                pltpu.sync_copy(zbuf, acc_sh.at[pl.ds(r0, NODE_CH)])
            plsc.subcore_barrier()

        sweep(u3_hbm)
        rescale(sy2_hbm)
        sweep(uw_hbm)
        rescale(sy1_hbm)
        sweep(uw_hbm)

        for q in range(NNCH):
            r0 = nbase + q * NODE_CH
            pltpu.sync_copy(acc_sh.at[pl.ds(r0, NODE_CH)], nu)
            pltpu.sync_copy(nu, m_hbm.at[pl.ds(c * NPAD + r0, NODE_CH)])

    return prop


_prop64 = _make_prop(64)
_prop32 = _make_prop(32)


_BN = 128
_GRID = NPAD // _BN


def _tc_prep_body(x_ref, w_ref, deg_ref, y0_ref, u3_ref, sy2_ref, sy1_ref,
                  s_ref, s2_ref):
    deg = deg_ref[0, :] + deg_ref[1, :]
    sv = jnp.where(deg > 0, lax.rsqrt(deg), 0.0)[:, None]
    y = jnp.dot(x_ref[...], w_ref[...], preferred_element_type=jnp.float32)
    y0_ref[...] = y[:, :HID]
    sy1_ref[...] = sv * y[:, HID:2 * HID]
    sy2_ref[...] = sv * y[:, 2 * HID:3 * HID]
    u3_ref[...] = sv * y[:, 3 * HID:]
    s_ref[...] = sv
    s2_ref[...] = sv * sv


def _tc_prep(xp, wcat, degpart):
    f32 = jnp.float32
    outs = (
        jax.ShapeDtypeStruct((NPAD, HID), f32),
        jax.ShapeDtypeStruct((NPAD, HID), f32),
        jax.ShapeDtypeStruct((NPAD, HID), f32),
        jax.ShapeDtypeStruct((NPAD, HID), f32),
        jax.ShapeDtypeStruct((NPAD, 1), f32),
        jax.ShapeDtypeStruct((NPAD, 1), f32),
    )
    blk = pl.BlockSpec((_BN, HID), lambda i: (i, 0))
    blk1 = pl.BlockSpec((_BN, 1), lambda i: (i, 0))
    return pl.pallas_call(
        _tc_prep_body,
        grid=(_GRID,),
        in_specs=[
            pl.BlockSpec((_BN, F_IN), lambda i: (i, 0)),
            pl.BlockSpec((F_IN, 4 * HID), lambda i: (0, 0)),
            pl.BlockSpec((NC, _BN), lambda i: (0, i)),
        ],
        out_specs=[blk, blk, blk, blk, blk1, blk1],
        out_shape=outs,
    )(xp, wcat, degpart)


def _tc_mid_body(m1_ref, y0_ref, s_ref, b1_ref, w_ref, z0_ref, u3_ref,
                 sz2_ref, sz1_ref):
    sv = s_ref[...]
    h = jnp.maximum(y0_ref[...] + sv * m1_ref[...] + b1_ref[...], 0.0)
    z = jnp.dot(h, w_ref[...], preferred_element_type=jnp.float32)
    z0_ref[...] = z[:, :NCLS]
    sz1_ref[...] = sv * z[:, NCLS:2 * NCLS]
    sz2_ref[...] = sv * z[:, 2 * NCLS:3 * NCLS]
    u3_ref[...] = sv * z[:, 3 * NCLS:]


def _tc_mid(m1, y0, svec, b1, wcat):
    f32 = jnp.float32
    outs = tuple(jax.ShapeDtypeStruct((NPAD, NCLS), f32) for _ in range(4))
    blk = pl.BlockSpec((_BN, NCLS), lambda i: (i, 0))
    return pl.pallas_call(
        _tc_mid_body,
        grid=(_GRID,),
        in_specs=[
            pl.BlockSpec((_BN, HID), lambda i: (i, 0)),
            pl.BlockSpec((_BN, HID), lambda i: (i, 0)),
            pl.BlockSpec((_BN, 1), lambda i: (i, 0)),
            pl.BlockSpec((1, HID), lambda i: (0, 0)),
            pl.BlockSpec((HID, 4 * NCLS), lambda i: (0, 0)),
        ],
        out_specs=[blk, blk, blk, blk],
        out_shape=outs,
    )(m1, y0, svec, b1, wcat)


def _tc_post_body(z0_ref, m2_ref, s_ref, b2_ref, out_ref):
    o = z0_ref[...] + s_ref[...] * m2_ref[...] + b2_ref[...]
    mx = jnp.max(o, axis=1, keepdims=True)
    ex = jnp.exp(o - mx)
    lse = mx + jnp.log(jnp.sum(ex, axis=1, keepdims=True))
    out_ref[...] = o - lse


def _tc_post(z0, m2, svec, b2):
    return pl.pallas_call(
        _tc_post_body,
        grid=(_GRID,),
        in_specs=[
            pl.BlockSpec((_BN, NCLS), lambda i: (i, 0)),
            pl.BlockSpec((_BN, NCLS), lambda i: (i, 0)),
            pl.BlockSpec((_BN, 1), lambda i: (i, 0)),
            pl.BlockSpec((1, NCLS), lambda i: (0, 0)),
        ],
        out_specs=pl.BlockSpec((_BN, NCLS), lambda i: (i, 0)),
        out_shape=jax.ShapeDtypeStruct((NPAD, NCLS), jnp.float32),
    )(z0, m2, svec, b2)


def _split_cols(a, w):
    return a.reshape(NPAD, 2, w).transpose(1, 0, 2).reshape(2 * NPAD, w)


def _merge_cols(a, w):
    return a.reshape(2, NPAD, w).transpose(1, 0, 2).reshape(NPAD, 2 * w)


def _fold(W):
    return jnp.concatenate(
        [W[0] - W[2], 3.0 * W[3] - W[1], 2.0 * W[2], -4.0 * W[3]], axis=1)


def kernel(x, edge_index, W1, b1, W2, b2):
    x = _f32(x)
    wcat1 = _fold(_f32(W1))
    wcat2 = _fold(_f32(W2))

    row = edge_index[0].astype(jnp.int32)
    col = edge_index[1].astype(jnp.int32)
    padv = (jnp.arange(E, E2, dtype=jnp.int32)) % N
    rowp = jnp.concatenate([row, padv]).reshape(E2 // 128, 128)
    colp_in = jnp.concatenate([col, padv]).reshape(E2 // 128, 128)
    rowoff = jnp.stack([rowp, rowp + NPAD])

    degpart, colp = _edge_prep(rowp, colp_in)
    degpart = degpart.reshape(NC, NPAD)

    xp = jnp.pad(x, ((0, NPAD - N), (0, 0)))
    y0, u3, sy2, sy1, svec, s2vec = _tc_prep(xp, wcat1, degpart)
    s2flat = s2vec.reshape(NPAD)

    m1s, _ = _prop64(rowoff, colp, _split_cols(u3, 64), _split_cols(sy2, 64),
                     _split_cols(sy1, 64), s2flat)
    m1 = _merge_cols(m1s, 64)

    z0, u3z, sz2, sz1 = _tc_mid(m1, y0, svec, b1.reshape(1, HID), wcat2)

    m2s, _ = _prop32(rowoff, colp, _split_cols(u3z, 32), _split_cols(sz2, 32),
                     _split_cols(sz1, 32), s2flat)
    m2 = _merge_cols(m2s, 32)

    out = _tc_post(z0, m2, svec, b2.reshape(1, NCLS))
    return out[:N]


# speedup vs baseline: 12.9366x; 1.0668x over previous
import functools

import jax
import jax.numpy as jnp
from jax import lax
from jax.experimental import pallas as pl
from jax.experimental.pallas import tpu as pltpu
from jax.experimental.pallas import tpu_sc as plsc

N = 10000
E = 320000
F_IN = 128
HID = 128
NCLS = 64

NC = 2
NS = 16
LANES = 16

NPAD = 10240
TRASHN = NPAD - N
E2 = 327680
EPT = E2 // NS
ECH = 128
NCHUNK = EPT // ECH
NBUF = 4
NGRP = NCHUNK // NBUF
GB = 2
NGRP2 = NCHUNK // GB
NPT = NPAD // NS
NODE_CH = 40
NNCH = NPT // NODE_CH
CRING = 2 * NBUF

EPW = E2 // (NC * NS)
PCH = 2048
PROWS = PCH // 128

_mesh = plsc.VectorSubcoreMesh(core_axis_name="c", subcore_axis_name="s")


def _f32(x):
    return jnp.asarray(x, jnp.float32)


@functools.partial(
    pl.kernel,
    out_type=(
        jax.ShapeDtypeStruct((NC * NPAD,), jnp.float32),
        jax.ShapeDtypeStruct((E2 // 128, 128), jnp.int32),
    ),
    mesh=_mesh,
    scratch_types=dict(
        rbuf=pltpu.VMEM((PROWS, 128), jnp.int32),
        cbuf=pltpu.VMEM((PROWS, 128), jnp.int32),
        cpbuf=pltpu.VMEM((PROWS, 128), jnp.int32),
        wbuf=pltpu.VMEM((PROWS, 128), jnp.float32),
        zbuf=pltpu.VMEM((NPT,), jnp.float32),
        degacc=pltpu.VMEM_SHARED((NPAD,), jnp.float32),
        sem=pltpu.SemaphoreType.DMA,
    ),
)
def _edge_prep(row_hbm, col_hbm, deg_hbm, colp_hbm, rbuf, cbuf, cpbuf, wbuf,
               zbuf, degacc, sem):
    c = lax.axis_index("c")
    s = lax.axis_index("s")
    wid = c * NS + s

    def _z(t, _):
        zbuf[pl.ds(t * LANES, LANES)] = jnp.zeros((LANES,), jnp.float32)
        return _
    lax.fori_loop(0, NPT // LANES, _z, None)
    pltpu.sync_copy(zbuf, degacc.at[pl.ds(s * NPT, NPT)])
    plsc.subcore_barrier()

    def chunk(ch, _):
        base = wid * (EPW // 128) + ch * PROWS
        pltpu.sync_copy(row_hbm.at[pl.ds(base, PROWS)], rbuf)
        pltpu.sync_copy(col_hbm.at[pl.ds(base, PROWS)], cbuf)

        def vec(t, _):
            j = t // (128 // LANES)
            k = t % (128 // LANES)
            r = rbuf[j, pl.ds(k * LANES, LANES)]
            cc = cbuf[j, pl.ds(k * LANES, LANES)]
            m = r != cc
            cpbuf[j, pl.ds(k * LANES, LANES)] = jnp.where(
                m, cc, N + jnp.remainder(cc, TRASHN))
            wbuf[j, pl.ds(k * LANES, LANES)] = jnp.where(m, 1.0, 0.0)
            return _
        lax.fori_loop(0, PROWS * (128 // LANES), vec, None)

        pltpu.sync_copy(cpbuf, colp_hbm.at[pl.ds(base, PROWS)])
        for j in range(PROWS):
            pltpu.sync_copy(wbuf.at[j], degacc.at[rbuf.at[j]], add=True)
        return _
    lax.fori_loop(0, EPW // PCH, chunk, None)
    plsc.subcore_barrier()

    pltpu.sync_copy(degacc.at[pl.ds(s * NPT, NPT)], zbuf)
    pltpu.sync_copy(zbuf, deg_hbm.at[pl.ds(c * NPAD + s * NPT, NPT)])


def _make_prop(W):
    QN = W // LANES

    @functools.partial(
        pl.kernel,
        out_type=(
            jax.ShapeDtypeStruct((NC * NPAD, W), jnp.float32),
            jax.ShapeDtypeStruct((NC * NPAD, W), jnp.float32),
        ),
        mesh=_mesh,
        scratch_types=dict(
            rixb=pltpu.VMEM((4, GB, ECH), jnp.int32),
            cixb=pltpu.VMEM((4, GB, ECH), jnp.int32),
            gbuf=pltpu.VMEM((2 * GB, ECH, W), jnp.float32),
            nsy=pltpu.VMEM((NODE_CH, W), jnp.float32),
            nu=pltpu.VMEM((NODE_CH, W), jnp.float32),
            zbuf=pltpu.VMEM((NODE_CH, W), jnp.float32),
            s2b=pltpu.VMEM((NPT,), jnp.float32),
            acc_sh=pltpu.VMEM_SHARED((NPAD, W), jnp.float32),
            gsem=pltpu.SemaphoreType.DMA((2 * GB,)),
            ssem=pltpu.SemaphoreType.DMA((2 * GB,)),
            risem=pltpu.SemaphoreType.DMA((4,)),
            cisem=pltpu.SemaphoreType.DMA((4,)),
            asem=pltpu.SemaphoreType.DMA,
            bsem=pltpu.SemaphoreType.DMA,
        ),
        compiler_params=pltpu.CompilerParams(needs_layout_passes=False,
                                             use_tc_tiling_on_sc=False),
    )
    def prop(rowoff_hbm, colp_hbm, u3_hbm, sy2_hbm, sy1_hbm, s2_hbm, m_hbm,
             uw_hbm, rixb, cixb, gbuf, nsy, nu, zbuf, s2b, acc_sh,
             gsem, ssem, risem, cisem, asem, bsem):
        c = lax.axis_index("c")
        s = lax.axis_index("s")
        nbase = s * NPT
        ebase = s * NCHUNK
        row_hbm = rowoff_hbm.at[c]

        pltpu.sync_copy(s2_hbm.at[pl.ds(nbase, NPT)], s2b)

        def _z(t, _):
            def _zrow(q, _2):
                zbuf[t, pl.ds(q * LANES, LANES)] = jnp.zeros((LANES,),
                                                             jnp.float32)
                return _2
            lax.fori_loop(0, QN, _zrow, None)
            return _
        lax.fori_loop(0, NODE_CH, _z, None)

        for q in range(NNCH):
            r0 = nbase + q * NODE_CH
            pltpu.sync_copy(zbuf, acc_sh.at[pl.ds(r0, NODE_CH)])
        plsc.subcore_barrier()

        def idx_load(g, p):
            pltpu.async_copy(row_hbm.at[pl.ds(ebase + g * GB, GB)],
                             rixb.at[p], risem.at[p])
            pltpu.async_copy(colp_hbm.at[pl.ds(ebase + g * GB, GB)],
                             cixb.at[p], cisem.at[p])

        def sweep(u_hbm):
            idx_load(0, 0)
            idx_load(1, 1)

            def do_group(g, k):
                p2 = k % 2
                p4 = k
                po = (k + 2) % 4
                rix = rixb.at[p4]
                cix = cixb.at[p4]
                pltpu.make_async_copy(
                    row_hbm.at[pl.ds(ebase + g * GB, GB)], rix,
                    risem.at[p4]).wait()

                @pl.when(g >= 2)
                def _():
                    for b in range(GB):
                        pltpu.make_async_copy(
                            gbuf.at[p2 * GB + b],
                            acc_sh.at[cixb.at[po].at[b]],
                            ssem.at[p2 * GB + b]).wait()

                @pl.when(g + 2 < NGRP2)
                def _():
                    idx_load(g + 2, po)

                pltpu.make_async_copy(
                    colp_hbm.at[pl.ds(ebase + g * GB, GB)], cix,
                    cisem.at[p4]).wait()
                for b in range(GB):
                    pltpu.async_copy(u_hbm.at[rix.at[b]],
                                     gbuf.at[p2 * GB + b],
                                     gsem.at[p2 * GB + b])
                for b in range(GB):
                    pltpu.make_async_copy(u_hbm.at[rix.at[b]],
                                          gbuf.at[p2 * GB + b],
                                          gsem.at[p2 * GB + b]).wait()
                    pltpu.async_copy(gbuf.at[p2 * GB + b],
                                     acc_sh.at[cix.at[b]],
                                     ssem.at[p2 * GB + b], add=True)

            def quad(it, _):
                for k in range(4):
                    do_group(4 * it + k, k)
                return _
            lax.fori_loop(0, NGRP2 // 4, quad, None)
            for k in (2, 3):
                p2 = k % 2
                for b in range(GB):
                    pltpu.make_async_copy(
                        gbuf.at[p2 * GB + b],
                        acc_sh.at[cixb.at[k].at[b]],
                        ssem.at[p2 * GB + b]).wait()
            plsc.subcore_barrier()

        def rescale(sy_hbm):
            for q in range(NNCH):
                r0 = nbase + q * NODE_CH
                pltpu.async_copy(acc_sh.at[pl.ds(r0, NODE_CH)], nu, asem)
                pltpu.async_copy(sy_hbm.at[pl.ds(c * NPAD + r0, NODE_CH)],
                                 nsy, bsem)
                pltpu.make_async_copy(acc_sh.at[pl.ds(r0, NODE_CH)], nu,
                                      asem).wait()
                pltpu.make_async_copy(sy_hbm.at[pl.ds(c * NPAD + r0, NODE_CH)],
                                      nsy, bsem).wait()

                def node(j, _):
                    jj = q * NODE_CH + j
                    s2v = plsc.load_gather(
                        s2b, [jnp.full((LANES,), jj, jnp.int32)])
                    for qq in range(QN):
                        sl = pl.ds(qq * LANES, LANES)
                        nu[j, sl] = nsy[j, sl] + s2v * nu[j, sl]
                    return _
                lax.fori_loop(0, NODE_CH, node, None)

                pltpu.sync_copy(nu, uw_hbm.at[pl.ds(c * NPAD + r0, NODE_CH)])
                pltpu.sync_copy(zbuf, acc_sh.at[pl.ds(r0, NODE_CH)])
            plsc.subcore_barrier()

        sweep(u3_hbm)
        rescale(sy2_hbm)
        sweep(uw_hbm)
        rescale(sy1_hbm)
        sweep(uw_hbm)

        for q in range(NNCH):
            r0 = nbase + q * NODE_CH
            pltpu.sync_copy(acc_sh.at[pl.ds(r0, NODE_CH)], nu)
            pltpu.sync_copy(nu, m_hbm.at[pl.ds(c * NPAD + r0, NODE_CH)])

    return prop


_prop64 = _make_prop(64)
_prop32 = _make_prop(32)


_BN = 128
_GRID = NPAD // _BN


def _tc_prep_body(x_ref, w_ref, deg_ref, y0_ref, u3_ref, sy2_ref, sy1_ref,
                  s_ref, s2_ref):
    deg = deg_ref[0, :] + deg_ref[1, :]
    sv = jnp.where(deg > 0, lax.rsqrt(deg), 0.0)[:, None]
    y = jnp.dot(x_ref[...], w_ref[...], preferred_element_type=jnp.float32)
    y0_ref[...] = y[:, :HID]
    sy1_ref[...] = sv * y[:, HID:2 * HID]
    sy2_ref[...] = sv * y[:, 2 * HID:3 * HID]
    u3_ref[...] = sv * y[:, 3 * HID:]
    s_ref[...] = sv
    s2_ref[...] = sv * sv


def _tc_prep(xp, wcat, degpart):
    f32 = jnp.float32
    outs = (
        jax.ShapeDtypeStruct((NPAD, HID), f32),
        jax.ShapeDtypeStruct((NPAD, HID), f32),
        jax.ShapeDtypeStruct((NPAD, HID), f32),
        jax.ShapeDtypeStruct((NPAD, HID), f32),
        jax.ShapeDtypeStruct((NPAD, 1), f32),
        jax.ShapeDtypeStruct((NPAD, 1), f32),
    )
    blk = pl.BlockSpec((_BN, HID), lambda i: (i, 0))
    blk1 = pl.BlockSpec((_BN, 1), lambda i: (i, 0))
    return pl.pallas_call(
        _tc_prep_body,
        grid=(_GRID,),
        in_specs=[
            pl.BlockSpec((_BN, F_IN), lambda i: (i, 0)),
            pl.BlockSpec((F_IN, 4 * HID), lambda i: (0, 0)),
            pl.BlockSpec((NC, _BN), lambda i: (0, i)),
        ],
        out_specs=[blk, blk, blk, blk, blk1, blk1],
        out_shape=outs,
    )(xp, wcat, degpart)


def _tc_mid_body(m1_ref, y0_ref, s_ref, b1_ref, w_ref, z0_ref, u3_ref,
                 sz2_ref, sz1_ref):
    sv = s_ref[...]
    h = jnp.maximum(y0_ref[...] + sv * m1_ref[...] + b1_ref[...], 0.0)
    z = jnp.dot(h, w_ref[...], preferred_element_type=jnp.float32)
    z0_ref[...] = z[:, :NCLS]
    sz1_ref[...] = sv * z[:, NCLS:2 * NCLS]
    sz2_ref[...] = sv * z[:, 2 * NCLS:3 * NCLS]
    u3_ref[...] = sv * z[:, 3 * NCLS:]


def _tc_mid(m1, y0, svec, b1, wcat):
    f32 = jnp.float32
    outs = tuple(jax.ShapeDtypeStruct((NPAD, NCLS), f32) for _ in range(4))
    blk = pl.BlockSpec((_BN, NCLS), lambda i: (i, 0))
    return pl.pallas_call(
        _tc_mid_body,
        grid=(_GRID,),
        in_specs=[
            pl.BlockSpec((_BN, HID), lambda i: (i, 0)),
            pl.BlockSpec((_BN, HID), lambda i: (i, 0)),
            pl.BlockSpec((_BN, 1), lambda i: (i, 0)),
            pl.BlockSpec((1, HID), lambda i: (0, 0)),
            pl.BlockSpec((HID, 4 * NCLS), lambda i: (0, 0)),
        ],
        out_specs=[blk, blk, blk, blk],
        out_shape=outs,
    )(m1, y0, svec, b1, wcat)


def _tc_post_body(z0_ref, m2_ref, s_ref, b2_ref, out_ref):
    o = z0_ref[...] + s_ref[...] * m2_ref[...] + b2_ref[...]
    mx = jnp.max(o, axis=1, keepdims=True)
    ex = jnp.exp(o - mx)
    lse = mx + jnp.log(jnp.sum(ex, axis=1, keepdims=True))
    out_ref[...] = o - lse


def _tc_post(z0, m2, svec, b2):
    return pl.pallas_call(
        _tc_post_body,
        grid=(_GRID,),
        in_specs=[
            pl.BlockSpec((_BN, NCLS), lambda i: (i, 0)),
            pl.BlockSpec((_BN, NCLS), lambda i: (i, 0)),
            pl.BlockSpec((_BN, 1), lambda i: (i, 0)),
            pl.BlockSpec((1, NCLS), lambda i: (0, 0)),
        ],
        out_specs=pl.BlockSpec((_BN, NCLS), lambda i: (i, 0)),
        out_shape=jax.ShapeDtypeStruct((NPAD, NCLS), jnp.float32),
    )(z0, m2, svec, b2)


def _split_cols(a, w):
    return a.reshape(NPAD, 2, w).transpose(1, 0, 2).reshape(2 * NPAD, w)


def _merge_cols(a, w):
    return a.reshape(2, NPAD, w).transpose(1, 0, 2).reshape(NPAD, 2 * w)


def _fold(W):
    return jnp.concatenate(
        [W[0] - W[2], 3.0 * W[3] - W[1], 2.0 * W[2], -4.0 * W[3]], axis=1)


def kernel(x, edge_index, W1, b1, W2, b2):
    x = _f32(x)
    wcat1 = _fold(_f32(W1))
    wcat2 = _fold(_f32(W2))

    row = edge_index[0].astype(jnp.int32)
    col = edge_index[1].astype(jnp.int32)
    padv = (jnp.arange(E, E2, dtype=jnp.int32)) % N
    rowp = jnp.concatenate([row, padv]).reshape(E2 // 128, 128)
    colp_in = jnp.concatenate([col, padv]).reshape(E2 // 128, 128)
    rowoff = jnp.stack([rowp, rowp + NPAD])

    degpart, colp = _edge_prep(rowp, colp_in)
    degpart = degpart.reshape(NC, NPAD)

    xp = jnp.pad(x, ((0, NPAD - N), (0, 0)))
    y0, u3, sy2, sy1, svec, s2vec = _tc_prep(xp, wcat1, degpart)
    s2flat = s2vec.reshape(NPAD)

    m1s, _ = _prop64(rowoff, colp, _split_cols(u3, 64), _split_cols(sy2, 64),
                     _split_cols(sy1, 64), s2flat)
    m1 = _merge_cols(m1s, 64)

    z0, u3z, sz2, sz1 = _tc_mid(m1, y0, svec, b1.reshape(1, HID), wcat2)

    m2s, _ = _prop32(rowoff, colp, _split_cols(u3z, 32), _split_cols(sz2, 32),
                     _split_cols(sz1, 32), s2flat)
    m2 = _merge_cols(m2s, 32)

    out = _tc_post(z0, m2, svec, b2.reshape(1, NCLS))
    return out[:N]
